# baseline (device time: 84340 ns/iter reference)
import jax
import jax.numpy as jnp
from jax import lax
from jax.experimental import pallas as pl
from jax.experimental.pallas import tpu as pltpu

N_DEV = 4


def kernel(A, B):
    m, _ = A.shape
    _, n = B.shape

    def body(a_ref, b_ref, out_ref, comm_ref, send_sems, recv_sems):
        my = lax.axis_index("i")
        left = (my - 1) % N_DEV
        right = (my + 1) % N_DEV

        barrier_sem = pltpu.get_barrier_semaphore()
        for nbr in (left, right):
            pl.semaphore_signal(
                barrier_sem, inc=1,
                device_id=(nbr,), device_id_type=pl.DeviceIdType.MESH,
            )
        pl.semaphore_wait(barrier_sem, 2)

        a_bf = a_ref[:, :].astype(jnp.bfloat16)
        b_bf = b_ref[:, :].astype(jnp.bfloat16)
        partial = jnp.dot(a_bf, b_bf, preferred_element_type=jnp.float32)
        out_ref[:, :] = partial
        comm_ref[0, :, :] = partial.astype(jnp.bfloat16)

        for h in range(N_DEV - 1):
            rdma = pltpu.make_async_remote_copy(
                src_ref=comm_ref.at[h],
                dst_ref=comm_ref.at[h + 1],
                send_sem=send_sems.at[h],
                recv_sem=recv_sems.at[h],
                device_id=(right,),
                device_id_type=pl.DeviceIdType.MESH,
            )
            rdma.start()
            rdma.wait()
            out_ref[:, :] += comm_ref[h + 1, :, :].astype(jnp.float32)

        z = out_ref[:, :]
        out_ref[:, :] = z * (1.0 / (1.0 + jnp.exp(-z)))

    return pl.pallas_call(
        body,
        out_shape=jax.ShapeDtypeStruct((m, n), jnp.float32),
        in_specs=[
            pl.BlockSpec(memory_space=pltpu.VMEM),
            pl.BlockSpec(memory_space=pltpu.VMEM),
        ],
        out_specs=pl.BlockSpec(memory_space=pltpu.VMEM),
        scratch_shapes=[
            pltpu.VMEM((N_DEV, m, n), jnp.bfloat16),
            pltpu.SemaphoreType.DMA((N_DEV - 1,)),
            pltpu.SemaphoreType.DMA((N_DEV - 1,)),
        ],
        compiler_params=pltpu.CompilerParams(collective_id=0),
    )(A, B)


# device time: 36456 ns/iter; 2.3135x vs baseline; 2.3135x over previous
import jax
import jax.numpy as jnp
from jax import lax
from jax.experimental import pallas as pl
from jax.experimental.pallas import tpu as pltpu

N_DEV = 4


def kernel(A, B):
    m, _ = A.shape
    _, n = B.shape
    nb = n // N_DEV

    def body(a_ref, b_ref, out_ref, part_ref, rs_recv, ag_recv,
             rs_send_sems, rs_recv_sems, ag_send_sems, ag_recv_sems):
        my = lax.axis_index("i")

        barrier_sem = pltpu.get_barrier_semaphore()
        for d in range(1, N_DEV):
            pl.semaphore_signal(
                barrier_sem, inc=1,
                device_id=((my + d) % N_DEV,),
                device_id_type=pl.DeviceIdType.MESH,
            )
        pl.semaphore_wait(barrier_sem, N_DEV - 1)

        a_bf = a_ref[:, :].astype(jnp.bfloat16)

        rs_rdmas = []
        for d in range(1, N_DEV):
            g = (my + d) % N_DEV
            b_blk = b_ref[:, pl.ds(g * nb, nb)].astype(jnp.bfloat16)
            part_ref[d, :, :] = jnp.dot(
                a_bf, b_blk, preferred_element_type=jnp.float32
            ).astype(jnp.bfloat16)
            rdma = pltpu.make_async_remote_copy(
                src_ref=part_ref.at[d],
                dst_ref=rs_recv.at[d - 1],
                send_sem=rs_send_sems.at[d - 1],
                recv_sem=rs_recv_sems.at[d - 1],
                device_id=(g,),
                device_id_type=pl.DeviceIdType.MESH,
            )
            rdma.start()
            rs_rdmas.append(rdma)

        b_own = b_ref[:, pl.ds(my * nb, nb)].astype(jnp.bfloat16)
        acc = jnp.dot(a_bf, b_own, preferred_element_type=jnp.float32)

        for r in rs_rdmas:
            r.wait_recv()
        acc = (acc
               + rs_recv[0, :, :].astype(jnp.float32)
               + rs_recv[1, :, :].astype(jnp.float32)
               + rs_recv[2, :, :].astype(jnp.float32))
        red = acc * (1.0 / (1.0 + jnp.exp(-acc)))
        out_ref[:, pl.ds(my * nb, nb)] = red
        part_ref[0, :, :] = red.astype(jnp.bfloat16)

        ag_rdmas = []
        for d in range(1, N_DEV):
            rdma = pltpu.make_async_remote_copy(
                src_ref=part_ref.at[0],
                dst_ref=ag_recv.at[d - 1],
                send_sem=ag_send_sems.at[d - 1],
                recv_sem=ag_recv_sems.at[d - 1],
                device_id=((my + d) % N_DEV,),
                device_id_type=pl.DeviceIdType.MESH,
            )
            rdma.start()
            ag_rdmas.append(rdma)

        for d in range(1, N_DEV):
            ag_rdmas[d - 1].wait_recv()
            src_dev = (my - d) % N_DEV
            out_ref[:, pl.ds(src_dev * nb, nb)] = (
                ag_recv[d - 1, :, :].astype(jnp.float32)
            )

        for r in rs_rdmas:
            r.wait_send()
        for r in ag_rdmas:
            r.wait_send()

    return pl.pallas_call(
        body,
        out_shape=jax.ShapeDtypeStruct((m, n), jnp.float32),
        in_specs=[
            pl.BlockSpec(memory_space=pltpu.VMEM),
            pl.BlockSpec(memory_space=pltpu.VMEM),
        ],
        out_specs=pl.BlockSpec(memory_space=pltpu.VMEM),
        scratch_shapes=[
            pltpu.VMEM((N_DEV, m, nb), jnp.bfloat16),
            pltpu.VMEM((N_DEV - 1, m, nb), jnp.bfloat16),
            pltpu.VMEM((N_DEV - 1, m, nb), jnp.bfloat16),
            pltpu.SemaphoreType.DMA((N_DEV - 1,)),
            pltpu.SemaphoreType.DMA((N_DEV - 1,)),
            pltpu.SemaphoreType.DMA((N_DEV - 1,)),
            pltpu.SemaphoreType.DMA((N_DEV - 1,)),
        ],
        compiler_params=pltpu.CompilerParams(collective_id=0),
    )(A, B)


# device time: 33878 ns/iter; 2.4895x vs baseline; 1.0761x over previous
import jax
import jax.numpy as jnp
from jax import lax
from jax.experimental import pallas as pl
from jax.experimental.pallas import tpu as pltpu

N_DEV = 4
S = 2


def kernel(A, B):
    m, _ = A.shape
    _, n = B.shape
    nb = n // N_DEV
    ns = nb // S

    def body(a_ref, b_ref, out_ref, part_ref, red_ref, rs_recv, ag_recv,
             rs_send_sems, rs_recv_sems, ag_send_sems, ag_recv_sems):
        my = lax.axis_index("i")

        barrier_sem = pltpu.get_barrier_semaphore()
        for d in range(1, N_DEV):
            pl.semaphore_signal(
                barrier_sem, inc=1,
                device_id=((my + d) % N_DEV,),
                device_id_type=pl.DeviceIdType.MESH,
            )

        a_bf = a_ref[:, :].astype(jnp.bfloat16)

        def rs_send(d, s):
            i = (d - 1) * S + s
            rdma = pltpu.make_async_remote_copy(
                src_ref=part_ref.at[d - 1, :, pl.ds(s * ns, ns)],
                dst_ref=rs_recv.at[d - 1, :, pl.ds(s * ns, ns)],
                send_sem=rs_send_sems.at[i],
                recv_sem=rs_recv_sems.at[i],
                device_id=((my + d) % N_DEV,),
                device_id_type=pl.DeviceIdType.MESH,
            )
            rdma.start()
            return rdma

        rs_rdmas = {}
        for d in range(1, N_DEV):
            g = (my + d) % N_DEV
            b_blk = b_ref[:, pl.ds(g * nb, nb)].astype(jnp.bfloat16)
            part_ref[d - 1, :, :] = jnp.dot(
                a_bf, b_blk, preferred_element_type=jnp.float32
            ).astype(jnp.bfloat16)
            if d == 1:
                pl.semaphore_wait(barrier_sem, N_DEV - 1)
            for s in range(S):
                rs_rdmas[(d, s)] = rs_send(d, s)

        b_own = b_ref[:, pl.ds(my * nb, nb)].astype(jnp.bfloat16)
        acc = jnp.dot(a_bf, b_own, preferred_element_type=jnp.float32)

        ag_rdmas = {}
        for s in range(S):
            for d in range(1, N_DEV):
                rs_rdmas[(d, s)].wait_recv()
            z = acc[:, s * ns:(s + 1) * ns]
            for k in range(N_DEV - 1):
                z = z + rs_recv[k, :, pl.ds(s * ns, ns)].astype(jnp.float32)
            z = z * (1.0 / (1.0 + jnp.exp(-z)))
            out_ref[:, pl.ds(my * nb + s * ns, ns)] = z
            red_ref[:, pl.ds(s * ns, ns)] = z.astype(jnp.bfloat16)
            for d in range(1, N_DEV):
                i = (d - 1) * S + s
                rdma = pltpu.make_async_remote_copy(
                    src_ref=red_ref.at[:, pl.ds(s * ns, ns)],
                    dst_ref=ag_recv.at[d - 1, :, pl.ds(s * ns, ns)],
                    send_sem=ag_send_sems.at[i],
                    recv_sem=ag_recv_sems.at[i],
                    device_id=((my + d) % N_DEV,),
                    device_id_type=pl.DeviceIdType.MESH,
                )
                rdma.start()
                ag_rdmas[(d, s)] = rdma

        for d in range(1, N_DEV):
            src_dev = (my - d) % N_DEV
            for s in range(S):
                ag_rdmas[(d, s)].wait_recv()
                out_ref[:, pl.ds(src_dev * nb + s * ns, ns)] = (
                    ag_recv[d - 1, :, pl.ds(s * ns, ns)].astype(jnp.float32)
                )

        for r in rs_rdmas.values():
            r.wait_send()
        for r in ag_rdmas.values():
            r.wait_send()

    return pl.pallas_call(
        body,
        out_shape=jax.ShapeDtypeStruct((m, n), jnp.float32),
        in_specs=[
            pl.BlockSpec(memory_space=pltpu.VMEM),
            pl.BlockSpec(memory_space=pltpu.VMEM),
        ],
        out_specs=pl.BlockSpec(memory_space=pltpu.VMEM),
        scratch_shapes=[
            pltpu.VMEM((N_DEV - 1, m, nb), jnp.bfloat16),
            pltpu.VMEM((m, nb), jnp.bfloat16),
            pltpu.VMEM((N_DEV - 1, m, nb), jnp.bfloat16),
            pltpu.VMEM((N_DEV - 1, m, nb), jnp.bfloat16),
            pltpu.SemaphoreType.DMA(((N_DEV - 1) * S,)),
            pltpu.SemaphoreType.DMA(((N_DEV - 1) * S,)),
            pltpu.SemaphoreType.DMA(((N_DEV - 1) * S,)),
            pltpu.SemaphoreType.DMA(((N_DEV - 1) * S,)),
        ],
        compiler_params=pltpu.CompilerParams(collective_id=0),
    )(A, B)
